# initial kernel scaffold (unmeasured)
import jax
import jax.numpy as jnp
from jax import lax
from jax.experimental import pallas as pl
from jax.experimental.pallas import tpu as pltpu


def kernel(
    x,
):
    def body(*refs):
        pass

    out_shape = jax.ShapeDtypeStruct(..., jnp.float32)
    return pl.pallas_call(body, out_shape=out_shape)(...)


# baseline (device time: 30708 ns/iter reference)
import jax
import jax.numpy as jnp
from jax import lax
from jax.experimental import pallas as pl
from jax.experimental.pallas import tpu as pltpu


def kernel(x):
    m_per, n = x.shape

    def body(x_ref, out_ref, send_sem, recv_sem):
        my_x = lax.axis_index("x")
        my_y = lax.axis_index("y")
        my_z = lax.axis_index("z")
        partner = (my_x, 1 - my_y, my_z)

        barrier_sem = pltpu.get_barrier_semaphore()
        pl.semaphore_signal(
            barrier_sem, inc=1,
            device_id=partner, device_id_type=pl.DeviceIdType.MESH,
        )
        pl.semaphore_wait(barrier_sem, 1)

        my_off = my_y * m_per
        out_ref[pl.ds(my_off, m_per), :] = x_ref[:, :].astype(jnp.bfloat16)

        rdma = pltpu.make_async_remote_copy(
            src_ref=out_ref.at[pl.ds(my_off, m_per), :],
            dst_ref=out_ref.at[pl.ds(my_off, m_per), :],
            send_sem=send_sem,
            recv_sem=recv_sem,
            device_id=partner,
            device_id_type=pl.DeviceIdType.MESH,
        )
        rdma.start()
        rdma.wait()

    return pl.pallas_call(
        body,
        out_shape=jax.ShapeDtypeStruct((2 * m_per, n), jnp.bfloat16),
        in_specs=[pl.BlockSpec(memory_space=pltpu.VMEM)],
        out_specs=pl.BlockSpec(memory_space=pltpu.VMEM),
        scratch_shapes=[
            pltpu.SemaphoreType.DMA,
            pltpu.SemaphoreType.DMA,
        ],
        compiler_params=pltpu.CompilerParams(collective_id=0),
    )(x)


# device time: 24380 ns/iter; 1.2596x vs baseline; 1.2596x over previous
import jax
import jax.numpy as jnp
from jax import lax
from jax.experimental import pallas as pl
from jax.experimental.pallas import tpu as pltpu

K = 4


def kernel(x):
    m_per, n = x.shape
    q_rows = m_per // 4
    c_rows = q_rows // K

    def body(x_ref, out_ref, y_s, y_r, x_s, x_r, z_s, z_r):
        my_x = lax.axis_index("x")
        my_y = lax.axis_index("y")
        my_z = lax.axis_index("z")
        yp = (my_x, 1 - my_y, my_z)
        xp = (1 - my_x, my_y, my_z)
        zp = (my_x, my_y, 1 - my_z)

        barrier = pltpu.get_barrier_semaphore()
        for nbr in (yp, xp, zp):
            pl.semaphore_signal(
                barrier, inc=1,
                device_id=nbr, device_id_type=pl.DeviceIdType.MESH,
            )
        pl.semaphore_wait(barrier, 3)

        q = 2 * my_x + my_z
        qx = 2 * (1 - my_x) + my_z
        qz = 2 * my_x + (1 - my_z)
        qd = 2 * (1 - my_x) + (1 - my_z)
        ob = my_y * m_per
        mb = (1 - my_y) * m_per

        def copy(rows, send_sem, recv_sem, dev):
            return pltpu.make_async_remote_copy(
                src_ref=out_ref.at[pl.ds(rows, c_rows), :],
                dst_ref=out_ref.at[pl.ds(rows, c_rows), :],
                send_sem=send_sem,
                recv_sem=recv_sem,
                device_id=dev,
                device_id_type=pl.DeviceIdType.MESH,
            )

        sends = []
        for j in range(4):
            qq = (q + j) % 4
            for k in range(K):
                src_r = qq * q_rows + k * c_rows
                out_ref[pl.ds(ob + src_r, c_rows), :] = (
                    x_ref[pl.ds(src_r, c_rows), :].astype(jnp.bfloat16)
                )
                if j == 0:
                    s = copy(ob + src_r, y_s.at[k], y_r.at[k], yp)
                    s.start()
                    sends.append(s)

        for k in range(K):
            rows = mb + q * q_rows + k * c_rows
            copy(rows, y_s.at[k], y_r.at[k], yp).wait_recv()
            for sem_s, sem_r, dev in ((x_s, x_r, xp), (z_s, z_r, zp)):
                s = copy(rows, sem_s.at[k], sem_r.at[k], dev)
                s.start()
                sends.append(s)

        for k in range(K):
            rows = mb + qx * q_rows + k * c_rows
            copy(rows, x_s.at[k], x_r.at[k], xp).wait_recv()
            s = copy(rows, z_s.at[K + k], z_r.at[K + k], zp)
            s.start()
            sends.append(s)

        for k in range(K):
            copy(mb + qz * q_rows + k * c_rows,
                 z_s.at[k], z_r.at[k], zp).wait_recv()
        for k in range(K):
            copy(mb + qd * q_rows + k * c_rows,
                 z_s.at[K + k], z_r.at[K + k], zp).wait_recv()

        for s in sends:
            s.wait_send()

    return pl.pallas_call(
        body,
        out_shape=jax.ShapeDtypeStruct((2 * m_per, n), jnp.bfloat16),
        in_specs=[pl.BlockSpec(memory_space=pltpu.VMEM)],
        out_specs=pl.BlockSpec(memory_space=pltpu.VMEM),
        scratch_shapes=[
            pltpu.SemaphoreType.DMA((K,)),
            pltpu.SemaphoreType.DMA((K,)),
            pltpu.SemaphoreType.DMA((K,)),
            pltpu.SemaphoreType.DMA((K,)),
            pltpu.SemaphoreType.DMA((2 * K,)),
            pltpu.SemaphoreType.DMA((2 * K,)),
        ],
        compiler_params=pltpu.CompilerParams(collective_id=0),
    )(x)


# device time: 24360 ns/iter; 1.2606x vs baseline; 1.0008x over previous
import jax
import jax.numpy as jnp
from jax import lax
from jax.experimental import pallas as pl
from jax.experimental.pallas import tpu as pltpu

K = 4


def kernel(x):
    m_per, n = x.shape
    q_rows = m_per // 4
    c_rows = q_rows // K

    def body(x_ref, out_ref, y_s, y_r, x_s, x_r, z_s, z_r):
        my_x = lax.axis_index("x")
        my_y = lax.axis_index("y")
        my_z = lax.axis_index("z")
        yp = (my_x, 1 - my_y, my_z)
        xp = (1 - my_x, my_y, my_z)
        zp = (my_x, my_y, 1 - my_z)

        q = 2 * my_x + my_z
        qx = 2 * (1 - my_x) + my_z
        qz = 2 * my_x + (1 - my_z)
        qd = 2 * (1 - my_x) + (1 - my_z)
        ob = my_y * m_per
        mb = (1 - my_y) * m_per

        barrier = pltpu.get_barrier_semaphore()
        for nbr in (yp, xp, zp):
            pl.semaphore_signal(
                barrier, inc=1,
                device_id=nbr, device_id_type=pl.DeviceIdType.MESH,
            )

        def convert(qq, k):
            r = qq * q_rows + k * c_rows
            out_ref[pl.ds(ob + r, c_rows), :] = (
                x_ref[pl.ds(r, c_rows), :].astype(jnp.bfloat16)
            )

        for k in range(K):
            convert(q, k)

        pl.semaphore_wait(barrier, 3)

        def copy(rows, send_sem, recv_sem, dev):
            return pltpu.make_async_remote_copy(
                src_ref=out_ref.at[pl.ds(rows, c_rows), :],
                dst_ref=out_ref.at[pl.ds(rows, c_rows), :],
                send_sem=send_sem,
                recv_sem=recv_sem,
                device_id=dev,
                device_id_type=pl.DeviceIdType.MESH,
            )

        sends = []
        for k in range(K):
            s = copy(ob + q * q_rows + k * c_rows, y_s.at[k], y_r.at[k], yp)
            s.start()
            sends.append(s)

        for k in range(K):
            rows = mb + q * q_rows + k * c_rows
            copy(rows, y_s.at[k], y_r.at[k], yp).wait_recv()
            for sem_s, sem_r, dev in ((x_s, x_r, xp), (z_s, z_r, zp)):
                s = copy(rows, sem_s.at[k], sem_r.at[k], dev)
                s.start()
                sends.append(s)
            convert((q + 1) % 4, k)

        for k in range(K):
            rows = mb + qx * q_rows + k * c_rows
            copy(rows, x_s.at[k], x_r.at[k], xp).wait_recv()
            s = copy(rows, z_s.at[K + k], z_r.at[K + k], zp)
            s.start()
            sends.append(s)
            convert((q + 2) % 4, k)

        for k in range(K):
            copy(mb + qz * q_rows + k * c_rows,
                 z_s.at[k], z_r.at[k], zp).wait_recv()
            convert((q + 3) % 4, k)
        for k in range(K):
            copy(mb + qd * q_rows + k * c_rows,
                 z_s.at[K + k], z_r.at[K + k], zp).wait_recv()

        for s in sends:
            s.wait_send()

    return pl.pallas_call(
        body,
        out_shape=jax.ShapeDtypeStruct((2 * m_per, n), jnp.bfloat16),
        in_specs=[pl.BlockSpec(memory_space=pltpu.VMEM)],
        out_specs=pl.BlockSpec(memory_space=pltpu.VMEM),
        scratch_shapes=[
            pltpu.SemaphoreType.DMA((K,)),
            pltpu.SemaphoreType.DMA((K,)),
            pltpu.SemaphoreType.DMA((K,)),
            pltpu.SemaphoreType.DMA((K,)),
            pltpu.SemaphoreType.DMA((2 * K,)),
            pltpu.SemaphoreType.DMA((2 * K,)),
        ],
        compiler_params=pltpu.CompilerParams(collective_id=0),
    )(x)


# device time: 20792 ns/iter; 1.4769x vs baseline; 1.1716x over previous
import jax
import jax.numpy as jnp
from jax import lax
from jax.experimental import pallas as pl
from jax.experimental.pallas import tpu as pltpu

KY = 4
KF = 2


def kernel(x):
    m_per, n = x.shape
    q_rows = m_per // 4
    c_rows = q_rows // KY
    f_rows = q_rows // KF

    def body(x_ref, out_ref, y_s, y_r, x_s, x_r, z_s, z_r):
        my_x = lax.axis_index("x")
        my_y = lax.axis_index("y")
        my_z = lax.axis_index("z")
        yp = (my_x, 1 - my_y, my_z)
        xp = (1 - my_x, my_y, my_z)
        zp = (my_x, my_y, 1 - my_z)

        q = 2 * my_x + my_z
        qd = 2 * (1 - my_x) + (1 - my_z)
        qx = 2 * (1 - my_x) + my_z
        qz = 2 * my_x + (1 - my_z)
        ob = my_y * m_per
        mb = (1 - my_y) * m_per

        barrier = pltpu.get_barrier_semaphore()
        for nbr in (yp, xp, zp):
            pl.semaphore_signal(
                barrier, inc=1,
                device_id=nbr, device_id_type=pl.DeviceIdType.MESH,
            )

        def convert(qq, k):
            r = qq * q_rows + k * c_rows
            out_ref[pl.ds(ob + r, c_rows), :] = (
                x_ref[pl.ds(r, c_rows), :].astype(jnp.bfloat16)
            )

        def copy(rows, nrows, send_sem, recv_sem, dev):
            return pltpu.make_async_remote_copy(
                src_ref=out_ref.at[pl.ds(rows, nrows), :],
                dst_ref=out_ref.at[pl.ds(rows, nrows), :],
                send_sem=send_sem,
                recv_sem=recv_sem,
                device_id=dev,
                device_id_type=pl.DeviceIdType.MESH,
            )

        convert(q, 0)
        pl.semaphore_wait(barrier, 3)

        sends = []
        for j, qq in enumerate((q, qd)):
            for k in range(KY):
                if (j, k) != (0, 0):
                    convert(qq, k)
                s = copy(ob + qq * q_rows + k * c_rows, c_rows,
                         y_s.at[j * KY + k], y_r.at[j * KY + k], yp)
                s.start()
                sends.append(s)

        for f in range(KF):
            for k in range(KY // KF):
                i = f * (KY // KF) + k
                copy(mb + q * q_rows + i * c_rows, c_rows,
                     y_s.at[i], y_r.at[i], yp).wait_recv()
            rows = mb + q * q_rows + f * f_rows
            for sem_s, sem_r, dev in ((x_s, x_r, xp), (z_s, z_r, zp)):
                s = copy(rows, f_rows, sem_s.at[f], sem_r.at[f], dev)
                s.start()
                sends.append(s)

        for qq in (qx, qz):
            for k in range(KY):
                convert(qq, k)

        for k in range(KY):
            copy(mb + qd * q_rows + k * c_rows, c_rows,
                 y_s.at[KY + k], y_r.at[KY + k], yp).wait_recv()
        for f in range(KF):
            copy(mb + qx * q_rows + f * f_rows, f_rows,
                 x_s.at[f], x_r.at[f], xp).wait_recv()
            copy(mb + qz * q_rows + f * f_rows, f_rows,
                 z_s.at[f], z_r.at[f], zp).wait_recv()

        for s in sends:
            s.wait_send()

    return pl.pallas_call(
        body,
        out_shape=jax.ShapeDtypeStruct((2 * m_per, n), jnp.bfloat16),
        in_specs=[pl.BlockSpec(memory_space=pltpu.VMEM)],
        out_specs=pl.BlockSpec(memory_space=pltpu.VMEM),
        scratch_shapes=[
            pltpu.SemaphoreType.DMA((2 * KY,)),
            pltpu.SemaphoreType.DMA((2 * KY,)),
            pltpu.SemaphoreType.DMA((KF,)),
            pltpu.SemaphoreType.DMA((KF,)),
            pltpu.SemaphoreType.DMA((KF,)),
            pltpu.SemaphoreType.DMA((KF,)),
        ],
        compiler_params=pltpu.CompilerParams(collective_id=0),
    )(x)
